# baseline (speedup 1.0000x reference)
import jax
import jax.numpy as jnp
from jax import lax
from jax.experimental import pallas as pl
from jax.experimental.pallas import tpu as pltpu
from jax.experimental.pallas import tpu_sc as plsc

N = 10000
E = 320000
D = 128
R = 8

NTILES = 32
EPT = E // NTILES
EPT1 = E // 16
C = 80
BLK = 400
NBLK = N // C


def _bcast_lane(v16, k):
    idx = jnp.full((16, 1), k, jnp.int32)
    dn = lax.GatherDimensionNumbers(
        offset_dims=(), collapsed_slice_dims=(0,), start_index_map=(0,))
    return lax.gather(v16, idx, dn, (1,),
                      mode=lax.GatherScatterMode.PROMISE_IN_BOUNDS)


def _iota16():
    return lax.iota(jnp.int32, 16)


def _mm_body(x_ref, wt_ref, o_ref):
    o_ref[...] = jnp.dot(x_ref[...], wt_ref[0],
                         preferred_element_type=jnp.float32)


def _matmul(x, wt):
    bn = 2000
    nb = N // bn
    return pl.pallas_call(
        _mm_body,
        grid=(R + 1, nb),
        in_specs=[
            pl.BlockSpec((bn, D), lambda r, i: (i, 0)),
            pl.BlockSpec((1, D, D), lambda r, i: (r, 0, 0)),
        ],
        out_specs=pl.BlockSpec((bn, D), lambda r, i: (r * nb + i, 0)),
        out_shape=jax.ShapeDtypeStruct(((R + 1) * N, D), jnp.float32),
    )(x, wt)


NR = N * R
CBLK = NR // NBLK


C1 = 128
NCH1 = EPT1 // C1
TAIL1 = EPT1 - NCH1 * C1
NCH2 = EPT // C


def _weights_body(src_hbm, dst_hbm, et, gidx_hbm, w_hbm, inv_hbm, widx_hbm,
                  cnt, invv, widxb, dstb1, typeb1, srcb1, didx, gb1, ones1,
                  didxt, typebt, srcbt, gbt, onest, buf, wbuf, stsem, scsem,
                  wrsem):
    cid = lax.axis_index("c")
    sid = lax.axis_index("s")
    wid = cid * 16 + sid

    def fill(i, _):
        ones1[pl.ds(i * 16, 16)] = jnp.full((16,), 1.0, jnp.float32)
        return 0
    lax.fori_loop(0, C1 // 16, fill, 0)

    def fillt(i, _):
        onest[pl.ds(i * 16, 16)] = jnp.full((16,), 1.0, jnp.float32)
        return 0
    lax.fori_loop(0, TAIL1 // 16, fillt, 0)

    def zero_buf(i, _):
        buf[pl.ds(i * 16, 16)] = jnp.zeros((16,), jnp.float32)
        return 0
    lax.fori_loop(0, CBLK // 16, zero_buf, 0)

    for it in range((NBLK + 15) // 16):
        b = it * 16 + sid

        @pl.when(b < NBLK)
        def _():
            pltpu.sync_copy(buf, cnt.at[pl.ds(b * CBLK, CBLK)])
    plsc.subcore_barrier()

    def stage1(j, p):
        base = sid * EPT1 + j * C1
        pltpu.async_copy(dst_hbm.at[pl.ds(base, C1)], dstb1.at[p],
                         stsem.at[p])
        pltpu.async_copy(et.at[pl.ds(base, C1)], typeb1.at[p], stsem.at[p])
        pltpu.async_copy(src_hbm.at[pl.ds(base, C1)], srcb1.at[p],
                         stsem.at[p])

    def wait1(j, p):
        base = sid * EPT1 + j * C1
        pltpu.make_async_copy(dst_hbm.at[pl.ds(base, C1)], dstb1.at[p],
                              stsem.at[p]).wait()
        pltpu.make_async_copy(et.at[pl.ds(base, C1)], typeb1.at[p],
                              stsem.at[p]).wait()
        pltpu.make_async_copy(src_hbm.at[pl.ds(base, C1)], srcb1.at[p],
                              stsem.at[p]).wait()

    def drain_add(p):
        pltpu.make_async_copy(ones1, cnt.at[didx.at[p]], scsem.at[p]).wait()

    def drain_wr1(j, p):
        base = sid * EPT1 + j * C1
        pltpu.make_async_copy(didx.at[p], widx_hbm.at[pl.ds(base, C1)],
                              wrsem.at[p]).wait()
        pltpu.make_async_copy(gb1.at[p], gidx_hbm.at[pl.ds(base, C1)],
                              wrsem.at[p]).wait()

    stage1(0, 0)

    def blk_body(j, _):
        p = lax.rem(j, 2)
        q = 1 - p

        @pl.when(j < NCH1 - 1)
        def _():
            stage1(j + 1, q)

        @pl.when(j > 1)
        def _():
            drain_add(p)
            drain_wr1(j - 2, p)
        wait1(j, p)
        base = sid * EPT1 + j * C1
        for g in range(C1 // 16):
            d = dstb1[p, pl.ds(g * 16, 16)]
            t = typeb1[p, pl.ds(g * 16, 16)]
            sv = srcb1[p, pl.ds(g * 16, 16)]
            didx[p, pl.ds(g * 16, 16)] = d * R + t
            gb1[p, pl.ds(g * 16, 16)] = (t + 1) * N + sv
        pltpu.async_copy(ones1, cnt.at[didx.at[p]], scsem.at[p], add=True)
        pltpu.async_copy(didx.at[p], widx_hbm.at[pl.ds(base, C1)],
                         wrsem.at[p])
        pltpu.async_copy(gb1.at[p], gidx_hbm.at[pl.ds(base, C1)],
                         wrsem.at[p])
        return 0
    lax.fori_loop(0, NCH1, blk_body, 0)
    drain_add(0)
    drain_add(1)
    drain_wr1(NCH1 - 2, (NCH1 - 2) % 2)
    drain_wr1(NCH1 - 1, (NCH1 - 1) % 2)
    tbase = sid * EPT1 + NCH1 * C1
    pltpu.sync_copy(dst_hbm.at[pl.ds(tbase, TAIL1)], didxt)
    pltpu.sync_copy(et.at[pl.ds(tbase, TAIL1)], typebt)
    pltpu.sync_copy(src_hbm.at[pl.ds(tbase, TAIL1)], srcbt)
    for g in range(TAIL1 // 16):
        d = didxt[pl.ds(g * 16, 16)]
        t = typebt[pl.ds(g * 16, 16)]
        sv = srcbt[pl.ds(g * 16, 16)]
        didxt[pl.ds(g * 16, 16)] = d * R + t
        gbt[pl.ds(g * 16, 16)] = (t + 1) * N + sv
    pltpu.sync_copy(onest, cnt.at[didxt], add=True)
    pltpu.sync_copy(didxt, widx_hbm.at[pl.ds(tbase, TAIL1)])
    pltpu.sync_copy(gbt, gidx_hbm.at[pl.ds(tbase, TAIL1)])
    plsc.subcore_barrier()

    for it in range((NBLK + 15) // 16):
        b = it * 16 + sid

        @pl.when(b < NBLK)
        def _():
            pltpu.sync_copy(cnt.at[pl.ds(b * CBLK, CBLK)], buf)

            def recip(i, _):
                v = buf[pl.ds(i * 16, 16)]
                buf[pl.ds(i * 16, 16)] = 1.0 / jnp.maximum(v, 1.0)
                return 0
            lax.fori_loop(0, CBLK // 16, recip, 0)
            pltpu.sync_copy(buf, inv_hbm.at[pl.ds(b * CBLK, CBLK)])
    plsc.subcore_barrier()

    pltpu.sync_copy(inv_hbm, invv)

    def stage2(j, p):
        base = wid * EPT + j * C
        pltpu.async_copy(widx_hbm.at[pl.ds(base, C)], widxb.at[p],
                         stsem.at[p])

    def wait2(j, p):
        base = wid * EPT + j * C
        pltpu.make_async_copy(widx_hbm.at[pl.ds(base, C)], widxb.at[p],
                              stsem.at[p]).wait()

    def drain_wr2(j, p):
        base = wid * EPT + j * C
        pltpu.make_async_copy(wbuf.at[p], w_hbm.at[pl.ds(base, C)],
                              wrsem.at[p]).wait()

    stage2(0, 0)

    def wblk_body(j, _):
        p = lax.rem(j, 2)
        q = 1 - p

        @pl.when(j < NCH2 - 1)
        def _():
            stage2(j + 1, q)

        @pl.when(j > 1)
        def _():
            drain_wr2(j - 2, p)
        wait2(j, p)
        base = wid * EPT + j * C
        for g in range(C // 16):
            ix = widxb[p, pl.ds(g * 16, 16)]
            wbuf[p, pl.ds(g * 16, 16)] = plsc.load_gather(invv, [ix])
        pltpu.async_copy(wbuf.at[p], w_hbm.at[pl.ds(base, C)], wrsem.at[p])
        return 0
    lax.fori_loop(0, NCH2, wblk_body, 0)
    drain_wr2(NCH2 - 2, (NCH2 - 2) % 2)
    drain_wr2(NCH2 - 1, (NCH2 - 1) % 2)


def _weights(src, dst, edge_type):
    mesh = plsc.VectorSubcoreMesh(core_axis_name="c", subcore_axis_name="s")
    f = pl.kernel(
        _weights_body,
        out_type=(jax.ShapeDtypeStruct((E,), jnp.int32),
                  jax.ShapeDtypeStruct((E,), jnp.float32),
                  jax.ShapeDtypeStruct((NR,), jnp.float32),
                  jax.ShapeDtypeStruct((E,), jnp.int32)),
        mesh=mesh,
        scratch_types=[
            pltpu.VMEM_SHARED((NR,), jnp.float32),
            pltpu.VMEM((NR,), jnp.float32),
            pltpu.VMEM((2, C), jnp.int32),
            pltpu.VMEM((2, C1), jnp.int32),
            pltpu.VMEM((2, C1), jnp.int32),
            pltpu.VMEM((2, C1), jnp.int32),
            pltpu.VMEM((2, C1), jnp.int32),
            pltpu.VMEM((2, C1), jnp.int32),
            pltpu.VMEM((C1,), jnp.float32),
            pltpu.VMEM((TAIL1,), jnp.int32),
            pltpu.VMEM((TAIL1,), jnp.int32),
            pltpu.VMEM((TAIL1,), jnp.int32),
            pltpu.VMEM((TAIL1,), jnp.int32),
            pltpu.VMEM((TAIL1,), jnp.float32),
            pltpu.VMEM((CBLK,), jnp.float32),
            pltpu.VMEM((2, C), jnp.float32),
            pltpu.SemaphoreType.DMA((2,)),
            pltpu.SemaphoreType.DMA((2,)),
            pltpu.SemaphoreType.DMA((2,)),
        ],
        compiler_params=pltpu.CompilerParams(needs_layout_passes=False),
    )
    return f(src, dst, edge_type)


def _agg_body(dst_hbm, gidx_hbm, w_hbm, h_hbm, part, acc, didx2, gidx, dstb,
              wb, rows, stsem, gsem, ssem):
    cid = lax.axis_index("c")
    sid = lax.axis_index("s")
    wid = cid * 16 + sid
    nchunk = EPT // C

    def zero_rows(i, _):
        for j in range(D // 16):
            rows[0, i, pl.ds(j * 16, 16)] = jnp.zeros((16,), jnp.float32)
        return 0
    lax.fori_loop(0, C, zero_rows, 0)

    for it in range((NBLK + 15) // 16):
        b = it * 16 + sid

        @pl.when(b < NBLK)
        def _():
            pltpu.sync_copy(rows.at[0], acc.at[pl.ds(b * C, C)])
    plsc.subcore_barrier()

    def stage(j, r):
        base = wid * EPT + j * C
        pltpu.async_copy(gidx_hbm.at[pl.ds(base, C)], gidx.at[r], stsem.at[r])
        pltpu.async_copy(dst_hbm.at[pl.ds(base, C)], dstb.at[r], stsem.at[r])
        pltpu.async_copy(w_hbm.at[pl.ds(base, C)], wb.at[r], stsem.at[r])

    def wait_stage(j, r):
        base = wid * EPT + j * C
        pltpu.make_async_copy(
            gidx_hbm.at[pl.ds(base, C)], gidx.at[r], stsem.at[r]).wait()
        pltpu.make_async_copy(
            dst_hbm.at[pl.ds(base, C)], dstb.at[r], stsem.at[r]).wait()
        pltpu.make_async_copy(
            w_hbm.at[pl.ds(base, C)], wb.at[r], stsem.at[r]).wait()

    def launch_gather(j):
        r = lax.rem(j, 4)
        pltpu.async_copy(h_hbm.at[gidx.at[r]], rows.at[r], gsem.at[r])

    def drain_scatter(j):
        r = lax.rem(j, 4)
        p = lax.rem(j, 2)
        pltpu.make_async_copy(rows.at[r], acc.at[didx2.at[p]],
                              ssem.at[p]).wait()

    stage(0, 0)
    stage(1, 1)
    stage(2, 2)
    wait_stage(0, 0)
    launch_gather(0)
    wait_stage(1, 1)
    launch_gather(1)

    def chunk(i, _):
        p = lax.rem(i, 2)
        r_i = lax.rem(i, 4)

        @pl.when(i >= 2)
        def _():
            drain_scatter(i - 2)

        @pl.when(i < nchunk - 2)
        def _():
            wait_stage(i + 2, lax.rem(i + 2, 4))
            launch_gather(i + 2)

        @pl.when(i < nchunk - 3)
        def _():
            stage(i + 3, lax.rem(i + 3, 4))

        pltpu.make_async_copy(
            h_hbm.at[gidx.at[r_i]], rows.at[r_i], gsem.at[r_i]).wait()
        for g in range(C // 16):
            didx2[p, pl.ds(g * 16, 16)] = dstb[r_i, pl.ds(g * 16, 16)]
            w16 = wb[r_i, pl.ds(g * 16, 16)]
            for k in range(16):
                wv = _bcast_lane(w16, k)
                r = g * 16 + k
                for j in range(D // 16):
                    rows[r_i, r, pl.ds(j * 16, 16)] = (
                        rows[r_i, r, pl.ds(j * 16, 16)] * wv)
        pltpu.async_copy(rows.at[r_i], acc.at[didx2.at[p]], ssem.at[p],
                         add=True)
        return 0
    lax.fori_loop(0, nchunk, chunk, 0)
    drain_scatter(nchunk - 2)
    drain_scatter(nchunk - 1)
    plsc.subcore_barrier()

    for it in range((NBLK + 15) // 16):
        b = it * 16 + sid

        @pl.when(b < NBLK)
        def _():
            pltpu.sync_copy(acc.at[pl.ds(b * C, C)], rows.at[0])
            pltpu.sync_copy(rows.at[0], part.at[cid, pl.ds(b * C, C)])


def _aggregate(dst, gidx, w, h_full):
    mesh = plsc.VectorSubcoreMesh(core_axis_name="c", subcore_axis_name="s")
    f = pl.kernel(
        _agg_body,
        out_type=jax.ShapeDtypeStruct((2, N, D), jnp.float32),
        mesh=mesh,
        scratch_types=[
            pltpu.VMEM_SHARED((N, D), jnp.float32),
            pltpu.VMEM((2, C), jnp.int32),
            pltpu.VMEM((4, C), jnp.int32),
            pltpu.VMEM((4, C), jnp.int32),
            pltpu.VMEM((4, C), jnp.float32),
            pltpu.VMEM((4, C, D), jnp.float32),
            pltpu.SemaphoreType.DMA((4,)),
            pltpu.SemaphoreType.DMA((4,)),
            pltpu.SemaphoreType.DMA((2,)),
        ],
        compiler_params=pltpu.CompilerParams(needs_layout_passes=False),
    )
    return f(dst, gidx, w, h_full)


def _comb_body(h_ref, p0_ref, p1_ref, b_ref, o_ref):
    o_ref[...] = h_ref[...] + p0_ref[0] + p1_ref[0] + b_ref[...]


def _combine(h_full, parts, bias):
    bn = 2000
    return pl.pallas_call(
        _comb_body,
        grid=(N // bn,),
        in_specs=[
            pl.BlockSpec((bn, D), lambda i: (i, 0)),
            pl.BlockSpec((1, bn, D), lambda i: (0, i, 0)),
            pl.BlockSpec((1, bn, D), lambda i: (1, i, 0)),
            pl.BlockSpec((1, D), lambda i: (0, 0)),
        ],
        out_specs=pl.BlockSpec((bn, D), lambda i: (i, 0)),
        out_shape=jax.ShapeDtypeStruct((N, D), jnp.float32),
    )(h_full, parts, parts, bias.reshape(1, D))


@jax.jit
def kernel(x, edge_index, edge_type, W_self, W_rel, bias):
    src = edge_index[0].astype(jnp.int32)
    dst = edge_index[1].astype(jnp.int32)
    edge_type = edge_type.astype(jnp.int32)
    wt = jnp.concatenate([W_self[None], W_rel], axis=0).transpose(0, 2, 1)
    h_full = _matmul(x, wt)
    gidx, w, _inv, _widx = _weights(src, dst, edge_type)
    parts = _aggregate(dst, gidx, w, h_full)
    return _combine(h_full, parts, bias)


# speedup vs baseline: 1.0052x; 1.0052x over previous
import jax
import jax.numpy as jnp
from jax import lax
from jax.experimental import pallas as pl
from jax.experimental.pallas import tpu as pltpu
from jax.experimental.pallas import tpu_sc as plsc

N = 10000
E = 320000
D = 128
R = 8

NTILES = 32
EPT = E // NTILES
EPT1 = E // 16
C = 80
BLK = 400
NBLK = N // C


def _bcast_lane(v16, k):
    idx = jnp.full((16, 1), k, jnp.int32)
    dn = lax.GatherDimensionNumbers(
        offset_dims=(), collapsed_slice_dims=(0,), start_index_map=(0,))
    return lax.gather(v16, idx, dn, (1,),
                      mode=lax.GatherScatterMode.PROMISE_IN_BOUNDS)


def _iota16():
    return lax.iota(jnp.int32, 16)


def _mm_body(x_ref, wt_ref, o_ref):
    o_ref[...] = jnp.dot(x_ref[...], wt_ref[0],
                         preferred_element_type=jnp.float32)


def _matmul(x, wt):
    bn = 2000
    nb = N // bn
    return pl.pallas_call(
        _mm_body,
        grid=(R + 1, nb),
        in_specs=[
            pl.BlockSpec((bn, D), lambda r, i: (i, 0)),
            pl.BlockSpec((1, D, D), lambda r, i: (r, 0, 0)),
        ],
        out_specs=pl.BlockSpec((bn, D), lambda r, i: (r * nb + i, 0)),
        out_shape=jax.ShapeDtypeStruct(((R + 1) * N, D), jnp.float32),
    )(x, wt)


NR = N * R
CBLK = NR // NBLK


C1 = 128
NCH1 = EPT1 // C1
TAIL1 = EPT1 - NCH1 * C1
NCH2 = EPT // C


def _weights_body(src_hbm, dst_hbm, et, gidx_hbm, w_hbm, inv_hbm, cnt, invv,
                  srcb, dstb, typeb, dstb1, typeb1, didx, ones1, didxt,
                  typebt, onest, buf, gbuf, wbuf, stsem, scsem, wrsem):
    cid = lax.axis_index("c")
    sid = lax.axis_index("s")
    wid = cid * 16 + sid

    def fill(i, _):
        ones1[pl.ds(i * 16, 16)] = jnp.full((16,), 1.0, jnp.float32)
        return 0
    lax.fori_loop(0, C1 // 16, fill, 0)

    def fillt(i, _):
        onest[pl.ds(i * 16, 16)] = jnp.full((16,), 1.0, jnp.float32)
        return 0
    lax.fori_loop(0, TAIL1 // 16, fillt, 0)

    def zero_buf(i, _):
        buf[pl.ds(i * 16, 16)] = jnp.zeros((16,), jnp.float32)
        return 0
    lax.fori_loop(0, CBLK // 16, zero_buf, 0)

    for it in range((NBLK + 15) // 16):
        b = it * 16 + sid

        @pl.when(b < NBLK)
        def _():
            pltpu.sync_copy(buf, cnt.at[pl.ds(b * CBLK, CBLK)])
    plsc.subcore_barrier()

    def stage1(j, p):
        base = sid * EPT1 + j * C1
        pltpu.async_copy(dst_hbm.at[pl.ds(base, C1)], dstb1.at[p],
                         stsem.at[p])
        pltpu.async_copy(et.at[pl.ds(base, C1)], typeb1.at[p], stsem.at[p])

    def wait1(j, p):
        base = sid * EPT1 + j * C1
        pltpu.make_async_copy(dst_hbm.at[pl.ds(base, C1)], dstb1.at[p],
                              stsem.at[p]).wait()
        pltpu.make_async_copy(et.at[pl.ds(base, C1)], typeb1.at[p],
                              stsem.at[p]).wait()

    def drain_add(p):
        pltpu.make_async_copy(ones1, cnt.at[didx.at[p]], scsem.at[p]).wait()

    stage1(0, 0)

    def blk_body(j, _):
        p = lax.rem(j, 2)
        q = 1 - p

        @pl.when(j < NCH1 - 1)
        def _():
            stage1(j + 1, q)

        @pl.when(j > 1)
        def _():
            drain_add(p)
        wait1(j, p)
        for g in range(C1 // 16):
            d = dstb1[p, pl.ds(g * 16, 16)]
            t = typeb1[p, pl.ds(g * 16, 16)]
            didx[p, pl.ds(g * 16, 16)] = d * R + t
        pltpu.async_copy(ones1, cnt.at[didx.at[p]], scsem.at[p], add=True)
        return 0
    lax.fori_loop(0, NCH1, blk_body, 0)
    drain_add(0)
    drain_add(1)
    tbase = sid * EPT1 + NCH1 * C1
    pltpu.sync_copy(dst_hbm.at[pl.ds(tbase, TAIL1)], didxt)
    pltpu.sync_copy(et.at[pl.ds(tbase, TAIL1)], typebt)
    for g in range(TAIL1 // 16):
        d = didxt[pl.ds(g * 16, 16)]
        t = typebt[pl.ds(g * 16, 16)]
        didxt[pl.ds(g * 16, 16)] = d * R + t
    pltpu.sync_copy(onest, cnt.at[didxt], add=True)
    plsc.subcore_barrier()

    for it in range((NBLK + 15) // 16):
        b = it * 16 + sid

        @pl.when(b < NBLK)
        def _():
            pltpu.sync_copy(cnt.at[pl.ds(b * CBLK, CBLK)], buf)

            def recip(i, _):
                v = buf[pl.ds(i * 16, 16)]
                buf[pl.ds(i * 16, 16)] = 1.0 / jnp.maximum(v, 1.0)
                return 0
            lax.fori_loop(0, CBLK // 16, recip, 0)
            pltpu.sync_copy(buf, inv_hbm.at[pl.ds(b * CBLK, CBLK)])
    plsc.subcore_barrier()

    pltpu.sync_copy(inv_hbm, invv)

    def stage2(j, p):
        base = wid * EPT + j * C
        pltpu.async_copy(src_hbm.at[pl.ds(base, C)], srcb.at[p], stsem.at[p])
        pltpu.async_copy(dst_hbm.at[pl.ds(base, C)], dstb.at[p], stsem.at[p])
        pltpu.async_copy(et.at[pl.ds(base, C)], typeb.at[p], stsem.at[p])

    def wait2(j, p):
        base = wid * EPT + j * C
        pltpu.make_async_copy(src_hbm.at[pl.ds(base, C)], srcb.at[p],
                              stsem.at[p]).wait()
        pltpu.make_async_copy(dst_hbm.at[pl.ds(base, C)], dstb.at[p],
                              stsem.at[p]).wait()
        pltpu.make_async_copy(et.at[pl.ds(base, C)], typeb.at[p],
                              stsem.at[p]).wait()

    def drain_wr(j, p):
        base = wid * EPT + j * C
        pltpu.make_async_copy(gbuf.at[p], gidx_hbm.at[pl.ds(base, C)],
                              wrsem.at[p]).wait()
        pltpu.make_async_copy(wbuf.at[p], w_hbm.at[pl.ds(base, C)],
                              wrsem.at[p]).wait()

    stage2(0, 0)

    def wblk_body(j, _):
        p = lax.rem(j, 2)
        q = 1 - p

        @pl.when(j < NCH2 - 1)
        def _():
            stage2(j + 1, q)

        @pl.when(j > 1)
        def _():
            drain_wr(j - 2, p)
        wait2(j, p)
        base = wid * EPT + j * C
        for g in range(C // 16):
            s = srcb[p, pl.ds(g * 16, 16)]
            d = dstb[p, pl.ds(g * 16, 16)]
            t = typeb[p, pl.ds(g * 16, 16)]
            gbuf[p, pl.ds(g * 16, 16)] = (t + 1) * N + s
            wbuf[p, pl.ds(g * 16, 16)] = plsc.load_gather(invv, [d * R + t])
        pltpu.async_copy(gbuf.at[p], gidx_hbm.at[pl.ds(base, C)], wrsem.at[p])
        pltpu.async_copy(wbuf.at[p], w_hbm.at[pl.ds(base, C)], wrsem.at[p])
        return 0
    lax.fori_loop(0, NCH2, wblk_body, 0)
    drain_wr(NCH2 - 2, (NCH2 - 2) % 2)
    drain_wr(NCH2 - 1, (NCH2 - 1) % 2)


def _weights(src, dst, edge_type):
    mesh = plsc.VectorSubcoreMesh(core_axis_name="c", subcore_axis_name="s")
    f = pl.kernel(
        _weights_body,
        out_type=(jax.ShapeDtypeStruct((E,), jnp.int32),
                  jax.ShapeDtypeStruct((E,), jnp.float32),
                  jax.ShapeDtypeStruct((NR,), jnp.float32)),
        mesh=mesh,
        scratch_types=[
            pltpu.VMEM_SHARED((NR,), jnp.float32),
            pltpu.VMEM((NR,), jnp.float32),
            pltpu.VMEM((2, C), jnp.int32),
            pltpu.VMEM((2, C), jnp.int32),
            pltpu.VMEM((2, C), jnp.int32),
            pltpu.VMEM((2, C1), jnp.int32),
            pltpu.VMEM((2, C1), jnp.int32),
            pltpu.VMEM((2, C1), jnp.int32),
            pltpu.VMEM((C1,), jnp.float32),
            pltpu.VMEM((TAIL1,), jnp.int32),
            pltpu.VMEM((TAIL1,), jnp.int32),
            pltpu.VMEM((TAIL1,), jnp.float32),
            pltpu.VMEM((CBLK,), jnp.float32),
            pltpu.VMEM((2, C), jnp.int32),
            pltpu.VMEM((2, C), jnp.float32),
            pltpu.SemaphoreType.DMA((2,)),
            pltpu.SemaphoreType.DMA((2,)),
            pltpu.SemaphoreType.DMA((2,)),
        ],
        compiler_params=pltpu.CompilerParams(needs_layout_passes=False),
    )
    return f(src, dst, edge_type)


def _agg_body(dst_hbm, gidx_hbm, w_hbm, h_hbm, part, acc, didx2, gidx, dstb,
              wb, rows, stsem, gsem, ssem):
    cid = lax.axis_index("c")
    sid = lax.axis_index("s")
    wid = cid * 16 + sid
    nchunk = EPT // C

    def zero_rows(i, _):
        for j in range(D // 16):
            rows[0, i, pl.ds(j * 16, 16)] = jnp.zeros((16,), jnp.float32)
        return 0
    lax.fori_loop(0, C, zero_rows, 0)

    for it in range((NBLK + 15) // 16):
        b = it * 16 + sid

        @pl.when(b < NBLK)
        def _():
            pltpu.sync_copy(rows.at[0], acc.at[pl.ds(b * C, C)])
    plsc.subcore_barrier()

    def stage(j, r):
        base = wid * EPT + j * C
        pltpu.async_copy(gidx_hbm.at[pl.ds(base, C)], gidx.at[r], stsem.at[r])
        pltpu.async_copy(dst_hbm.at[pl.ds(base, C)], dstb.at[r], stsem.at[r])
        pltpu.async_copy(w_hbm.at[pl.ds(base, C)], wb.at[r], stsem.at[r])

    def wait_stage(j, r):
        base = wid * EPT + j * C
        pltpu.make_async_copy(
            gidx_hbm.at[pl.ds(base, C)], gidx.at[r], stsem.at[r]).wait()
        pltpu.make_async_copy(
            dst_hbm.at[pl.ds(base, C)], dstb.at[r], stsem.at[r]).wait()
        pltpu.make_async_copy(
            w_hbm.at[pl.ds(base, C)], wb.at[r], stsem.at[r]).wait()

    def launch_gather(j):
        r = lax.rem(j, 4)
        pltpu.async_copy(h_hbm.at[gidx.at[r]], rows.at[r], gsem.at[r])

    def drain_scatter(j):
        r = lax.rem(j, 4)
        p = lax.rem(j, 2)
        pltpu.make_async_copy(rows.at[r], acc.at[didx2.at[p]],
                              ssem.at[p]).wait()

    stage(0, 0)
    stage(1, 1)
    stage(2, 2)
    wait_stage(0, 0)
    launch_gather(0)
    wait_stage(1, 1)
    launch_gather(1)

    def chunk(i, _):
        p = lax.rem(i, 2)
        r_i = lax.rem(i, 4)

        @pl.when(i >= 2)
        def _():
            drain_scatter(i - 2)

        @pl.when(i < nchunk - 2)
        def _():
            wait_stage(i + 2, lax.rem(i + 2, 4))
            launch_gather(i + 2)

        @pl.when(i < nchunk - 3)
        def _():
            stage(i + 3, lax.rem(i + 3, 4))

        pltpu.make_async_copy(
            h_hbm.at[gidx.at[r_i]], rows.at[r_i], gsem.at[r_i]).wait()
        for g in range(C // 16):
            didx2[p, pl.ds(g * 16, 16)] = dstb[r_i, pl.ds(g * 16, 16)]
            w16 = wb[r_i, pl.ds(g * 16, 16)]
            for k in range(16):
                wv = _bcast_lane(w16, k)
                r = g * 16 + k
                for j in range(D // 16):
                    rows[r_i, r, pl.ds(j * 16, 16)] = (
                        rows[r_i, r, pl.ds(j * 16, 16)] * wv)
        pltpu.async_copy(rows.at[r_i], acc.at[didx2.at[p]], ssem.at[p],
                         add=True)
        return 0
    lax.fori_loop(0, nchunk, chunk, 0)
    drain_scatter(nchunk - 2)
    drain_scatter(nchunk - 1)
    plsc.subcore_barrier()

    for it in range((NBLK + 15) // 16):
        b = it * 16 + sid

        @pl.when(b < NBLK)
        def _():
            pltpu.sync_copy(acc.at[pl.ds(b * C, C)], rows.at[0])
            pltpu.sync_copy(rows.at[0], part.at[cid, pl.ds(b * C, C)])


def _aggregate(dst, gidx, w, h_full):
    mesh = plsc.VectorSubcoreMesh(core_axis_name="c", subcore_axis_name="s")
    f = pl.kernel(
        _agg_body,
        out_type=jax.ShapeDtypeStruct((2, N, D), jnp.float32),
        mesh=mesh,
        scratch_types=[
            pltpu.VMEM_SHARED((N, D), jnp.float32),
            pltpu.VMEM((2, C), jnp.int32),
            pltpu.VMEM((4, C), jnp.int32),
            pltpu.VMEM((4, C), jnp.int32),
            pltpu.VMEM((4, C), jnp.float32),
            pltpu.VMEM((4, C, D), jnp.float32),
            pltpu.SemaphoreType.DMA((4,)),
            pltpu.SemaphoreType.DMA((4,)),
            pltpu.SemaphoreType.DMA((2,)),
        ],
        compiler_params=pltpu.CompilerParams(needs_layout_passes=False),
    )
    return f(dst, gidx, w, h_full)


def _comb_body(h_ref, p0_ref, p1_ref, b_ref, o_ref):
    o_ref[...] = h_ref[...] + p0_ref[0] + p1_ref[0] + b_ref[...]


def _combine(h_full, parts, bias):
    bn = 2000
    return pl.pallas_call(
        _comb_body,
        grid=(N // bn,),
        in_specs=[
            pl.BlockSpec((bn, D), lambda i: (i, 0)),
            pl.BlockSpec((1, bn, D), lambda i: (0, i, 0)),
            pl.BlockSpec((1, bn, D), lambda i: (1, i, 0)),
            pl.BlockSpec((1, D), lambda i: (0, 0)),
        ],
        out_specs=pl.BlockSpec((bn, D), lambda i: (i, 0)),
        out_shape=jax.ShapeDtypeStruct((N, D), jnp.float32),
    )(h_full, parts, parts, bias.reshape(1, D))


@jax.jit
def kernel(x, edge_index, edge_type, W_self, W_rel, bias):
    src = edge_index[0].astype(jnp.int32)
    dst = edge_index[1].astype(jnp.int32)
    edge_type = edge_type.astype(jnp.int32)
    wt = jnp.concatenate([W_self[None], W_rel], axis=0).transpose(0, 2, 1)
    h_full = _matmul(x, wt)
    gidx, w, _inv = _weights(src, dst, edge_type)
    parts = _aggregate(dst, gidx, w, h_full)
    return _combine(h_full, parts, bias)
